# baseline (speedup 1.0000x reference)
import functools

import jax
import jax.numpy as jnp
from jax import lax
from jax.experimental import pallas as pl
from jax.experimental.pallas import tpu as pltpu
from jax.experimental.pallas import tpu_sc as plsc

N_CORES = 2
N_SUBCORES = 16
N_WORKERS = N_CORES * N_SUBCORES
CHUNK = 128
LANES = 16


def _make_bpr_kernel(B, D, N):
    rows_per_w = B // N_WORKERS
    n_chunks = rows_per_w // CHUNK
    n_groups = rows_per_w // LANES
    mesh = plsc.VectorSubcoreMesh(core_axis_name="c", subcore_axis_name="s")

    @functools.partial(
        pl.kernel,
        mesh=mesh,
        compiler_params=pltpu.CompilerParams(
            needs_layout_passes=False, use_tc_tiling_on_sc=False),
        out_type=(
            jax.ShapeDtypeStruct((B,), jnp.float32),
            jax.ShapeDtypeStruct((B,), jnp.float32),
        ),
        scratch_types=[
            pltpu.VMEM((rows_per_w,), jnp.int32),
            pltpu.VMEM((rows_per_w,), jnp.int32),
            pltpu.VMEM((rows_per_w,), jnp.int32),
            pltpu.VMEM((D, rows_per_w), jnp.float32),
            pltpu.VMEM((D, rows_per_w), jnp.float32),
            pltpu.VMEM((D, rows_per_w), jnp.float32),
            pltpu.VMEM((rows_per_w,), jnp.float32),
            pltpu.VMEM((rows_per_w,), jnp.float32),
            pltpu.SemaphoreType.DMA,
        ],
    )
    def kern(uT_hbm, iT_hbm, u_hbm, i_hbm, j_hbm,
             pos_hbm, neg_hbm,
             u_v, i_v, j_v, ue_v, ie_v, je_v, pos_v, neg_v, sem):
        wid = lax.axis_index("s") * N_CORES + lax.axis_index("c")
        base = wid * rows_per_w

        pltpu.sync_copy(u_hbm.at[wid], u_v)
        pltpu.sync_copy(i_hbm.at[wid], i_v)
        pltpu.sync_copy(j_hbm.at[wid], j_v)

        copies = []
        for c in range(n_chunks):
            sl = pl.ds(c * CHUNK, CHUNK)
            for d in range(D):
                col = pl.ds(d * N, N)
                copies.append(pltpu.async_copy(
                    uT_hbm.at[col].at[u_v.at[sl]], ue_v.at[d, sl], sem))
                copies.append(pltpu.async_copy(
                    iT_hbm.at[col].at[i_v.at[sl]], ie_v.at[d, sl], sem))
                copies.append(pltpu.async_copy(
                    iT_hbm.at[col].at[j_v.at[sl]], je_v.at[d, sl], sem))
        for cp in copies:
            cp.wait()

        def group_body(g, carry):
            sl = pl.ds(g * LANES, LANES)
            acc_p = jnp.zeros((LANES,), jnp.float32)
            acc_n = jnp.zeros((LANES,), jnp.float32)
            for d in range(D):
                ue = ue_v[d, sl]
                acc_p = acc_p + ue * ie_v[d, sl]
                acc_n = acc_n + ue * je_v[d, sl]
            pos_v[sl] = acc_p
            neg_v[sl] = acc_n
            return carry

        lax.fori_loop(0, n_groups, group_body, 0)

        pltpu.sync_copy(pos_v, pos_hbm.at[pl.ds(base, rows_per_w)])
        pltpu.sync_copy(neg_v, neg_hbm.at[pl.ds(base, rows_per_w)])

    return kern


def kernel(u, i, j, labels, user_embed, item_embed):
    B = u.shape[0]
    N, D = user_embed.shape
    uT = user_embed.T.reshape(-1)
    iT = item_embed.T.reshape(-1)
    u32 = u.astype(jnp.int32).reshape(N_WORKERS, -1)
    i32 = i.astype(jnp.int32).reshape(N_WORKERS, -1)
    j32 = j.astype(jnp.int32).reshape(N_WORKERS, -1)
    pos, neg = _make_bpr_kernel(B, D, N)(uT, iT, u32, i32, j32)
    return pos.reshape(B, 1), neg.reshape(B, 1)


# speedup vs baseline: 4.7760x; 4.7760x over previous
import functools

import jax
import jax.numpy as jnp
from jax import lax
from jax.experimental import pallas as pl
from jax.experimental.pallas import tpu as pltpu
from jax.experimental.pallas import tpu_sc as plsc

N_CORES = 2
N_SUBCORES = 16
N_WORKERS = N_CORES * N_SUBCORES
CHUNK = 128
LANES = 16


def _make_bpr_kernel(B, D):
    rows_per_w = B // N_WORKERS
    n_chunks = rows_per_w // CHUNK
    n_groups = rows_per_w // LANES
    mesh = plsc.VectorSubcoreMesh(core_axis_name="c", subcore_axis_name="s")

    @functools.partial(
        pl.kernel,
        mesh=mesh,
        compiler_params=pltpu.CompilerParams(
            needs_layout_passes=False, use_tc_tiling_on_sc=False),
        out_type=(
            jax.ShapeDtypeStruct((B,), jnp.float32),
            jax.ShapeDtypeStruct((B,), jnp.float32),
        ),
        scratch_types=[
            pltpu.VMEM((n_chunks, CHUNK), jnp.int32),
            pltpu.VMEM((n_chunks, CHUNK), jnp.int32),
            pltpu.VMEM((n_chunks, CHUNK), jnp.int32),
            pltpu.VMEM((rows_per_w, D), jnp.bfloat16),
            pltpu.VMEM((rows_per_w, D), jnp.bfloat16),
            pltpu.VMEM((rows_per_w, D), jnp.bfloat16),
            pltpu.VMEM((rows_per_w, D), jnp.float32),
            pltpu.VMEM((rows_per_w, D), jnp.float32),
            pltpu.VMEM((rows_per_w, D), jnp.float32),
            pltpu.VMEM((rows_per_w,), jnp.float32),
            pltpu.VMEM((rows_per_w,), jnp.float32),
            pltpu.SemaphoreType.DMA,
        ],
    )
    def kern(user_hbm, item_hbm, u_hbm, i_hbm, j_hbm,
             pos_hbm, neg_hbm,
             u_v, i_v, j_v, ue16_v, ie16_v, je16_v,
             ue_v, ie_v, je_v, pos_v, neg_v, sem):
        wid = lax.axis_index("s") * N_CORES + lax.axis_index("c")
        base = wid * rows_per_w

        pltpu.sync_copy(u_hbm.at[wid], u_v)
        pltpu.sync_copy(i_hbm.at[wid], i_v)
        pltpu.sync_copy(j_hbm.at[wid], j_v)

        copies = []
        for c in range(n_chunks):
            sl = pl.ds(c * CHUNK, CHUNK)
            copies.append(pltpu.async_copy(user_hbm.at[u_v.at[c]], ue16_v.at[sl], sem))
            copies.append(pltpu.async_copy(item_hbm.at[i_v.at[c]], ie16_v.at[sl], sem))
            copies.append(pltpu.async_copy(item_hbm.at[j_v.at[c]], je16_v.at[sl], sem))
        for cp in copies:
            cp.wait()

        lo = pl.ds(0, LANES)
        hi = pl.ds(LANES, LANES)

        def unpack_body(r, carry):
            for src_v, dst_v in ((ue16_v, ue_v), (ie16_v, ie_v), (je16_v, je_v)):
                row = src_v[r, :]
                a, b = plsc.unpack(row, format=plsc.PackFormat.INTERLEAVED)
                dst_v[r, lo] = a
                dst_v[r, hi] = b
            return carry

        lax.fori_loop(0, rows_per_w, unpack_body, 0)

        lanes = lax.iota(jnp.int32, LANES)

        def group_body(g, carry):
            rows = g * LANES + lanes
            acc_p = jnp.zeros((LANES,), jnp.float32)
            acc_n = jnp.zeros((LANES,), jnp.float32)
            for d in range(D):
                col = jnp.full((LANES,), d, jnp.int32)
                ue = plsc.load_gather(ue_v, [rows, col])
                ie = plsc.load_gather(ie_v, [rows, col])
                je = plsc.load_gather(je_v, [rows, col])
                acc_p = acc_p + ue * ie
                acc_n = acc_n + ue * je
            pos_v[pl.ds(g * LANES, LANES)] = acc_p
            neg_v[pl.ds(g * LANES, LANES)] = acc_n
            return carry

        lax.fori_loop(0, n_groups, group_body, 0)

        pltpu.sync_copy(pos_v, pos_hbm.at[pl.ds(base, rows_per_w)])
        pltpu.sync_copy(neg_v, neg_hbm.at[pl.ds(base, rows_per_w)])

    return kern


def kernel(u, i, j, labels, user_embed, item_embed):
    B = u.shape[0]
    D = user_embed.shape[1]
    u16 = user_embed.astype(jnp.bfloat16)
    i16 = item_embed.astype(jnp.bfloat16)
    u32 = u.astype(jnp.int32).reshape(N_WORKERS, -1, CHUNK)
    i32 = i.astype(jnp.int32).reshape(N_WORKERS, -1, CHUNK)
    j32 = j.astype(jnp.int32).reshape(N_WORKERS, -1, CHUNK)
    pos, neg = _make_bpr_kernel(B, D)(u16, i16, u32, i32, j32)
    return pos.reshape(B, 1), neg.reshape(B, 1)


# speedup vs baseline: 5.6151x; 1.1757x over previous
import functools

import jax
import jax.numpy as jnp
from jax import lax
from jax.experimental import pallas as pl
from jax.experimental.pallas import tpu as pltpu
from jax.experimental.pallas import tpu_sc as plsc

N_CORES = 2
N_SUBCORES = 16
N_WORKERS = N_CORES * N_SUBCORES
CHUNK = 128
LANES = 16


def _make_bpr_kernel(B, D):
    rows_per_w = B // N_WORKERS
    n_chunks = rows_per_w // CHUNK
    n_groups = rows_per_w // LANES
    mesh = plsc.VectorSubcoreMesh(core_axis_name="c", subcore_axis_name="s")

    @functools.partial(
        pl.kernel,
        mesh=mesh,
        compiler_params=pltpu.CompilerParams(
            needs_layout_passes=False, use_tc_tiling_on_sc=False),
        out_type=(
            jax.ShapeDtypeStruct((B,), jnp.float32),
            jax.ShapeDtypeStruct((B,), jnp.float32),
        ),
        scratch_types=[
            pltpu.VMEM((n_chunks, CHUNK), jnp.int32),
            pltpu.VMEM((n_chunks, CHUNK), jnp.int32),
            pltpu.VMEM((n_chunks, CHUNK), jnp.int32),
            pltpu.VMEM((rows_per_w, D), jnp.float32),
            pltpu.VMEM((rows_per_w, D), jnp.float32),
            pltpu.VMEM((rows_per_w, D), jnp.float32),
            pltpu.VMEM((rows_per_w,), jnp.float32),
            pltpu.VMEM((rows_per_w,), jnp.float32),
            pltpu.SemaphoreType.DMA,
        ],
    )
    def kern(user_hbm, item_hbm, u_hbm, i_hbm, j_hbm,
             pos_hbm, neg_hbm,
             u_v, i_v, j_v, ue_v, ie_v, je_v, pos_v, neg_v, sem):
        wid = lax.axis_index("s") * N_CORES + lax.axis_index("c")
        base = wid * rows_per_w

        pltpu.sync_copy(u_hbm.at[wid], u_v)
        pltpu.sync_copy(i_hbm.at[wid], i_v)
        pltpu.sync_copy(j_hbm.at[wid], j_v)

        copies = []
        for c in range(n_chunks):
            sl = pl.ds(c * CHUNK, CHUNK)
            copies.append(pltpu.async_copy(user_hbm.at[u_v.at[c]], ue_v.at[sl], sem))
            copies.append(pltpu.async_copy(item_hbm.at[i_v.at[c]], ie_v.at[sl], sem))
            copies.append(pltpu.async_copy(item_hbm.at[j_v.at[c]], je_v.at[sl], sem))
        for cp in copies:
            cp.wait()

        lanes = lax.iota(jnp.int32, LANES)

        def group_body(g, carry):
            rows = g * LANES + lanes
            acc_p = jnp.zeros((LANES,), jnp.float32)
            acc_n = jnp.zeros((LANES,), jnp.float32)
            for d in range(D):
                col = jnp.full((LANES,), d, jnp.int32)
                ue = plsc.load_gather(ue_v, [rows, col])
                ie = plsc.load_gather(ie_v, [rows, col])
                je = plsc.load_gather(je_v, [rows, col])
                acc_p = acc_p + ue * ie
                acc_n = acc_n + ue * je
            pos_v[pl.ds(g * LANES, LANES)] = acc_p
            neg_v[pl.ds(g * LANES, LANES)] = acc_n
            return carry

        lax.fori_loop(0, n_groups, group_body, 0)

        pltpu.sync_copy(pos_v, pos_hbm.at[pl.ds(base, rows_per_w)])
        pltpu.sync_copy(neg_v, neg_hbm.at[pl.ds(base, rows_per_w)])

    return kern


def kernel(u, i, j, labels, user_embed, item_embed):
    B = u.shape[0]
    D = user_embed.shape[1]
    u32 = u.astype(jnp.int32).reshape(N_WORKERS, -1, CHUNK)
    i32 = i.astype(jnp.int32).reshape(N_WORKERS, -1, CHUNK)
    j32 = j.astype(jnp.int32).reshape(N_WORKERS, -1, CHUNK)
    pos, neg = _make_bpr_kernel(B, D)(user_embed, item_embed, u32, i32, j32)
    return pos.reshape(B, 1), neg.reshape(B, 1)
